# Initial kernel scaffold; baseline (speedup 1.0000x reference)
import jax
import jax.numpy as jnp
from jax.experimental import pallas as pl


def kernel(x, edge_index, graph_ids, graph_labels, W1, b1, W2, b2):
    raise NotImplementedError("write your pallas kernel here")


# speedup vs baseline: 9.4794x; 9.4794x over previous
import jax
import jax.numpy as jnp
from jax import lax
from jax.experimental import pallas as pl
from jax.experimental.pallas import tpu as pltpu
from jax.experimental.pallas import tpu_sc as plsc

N = 10000
E = 320000
D = 128
G = 200
C = 5

NC = 2
NS = 16
NW = NC * NS
CHUNK = 128
NP = 10240
EPT = 10240
GCH = 8
GROUPS = EPT // (GCH * CHUNK)
EP = NW * EPT
ROWS_PER_TILE = NP // NS


def _edge_agg_body(table_hbm, zeros_hbm, src_hbm, dst_hbm, out_hbm,
                   srcv, dstv, rows0, rows1, acc, sem0, sem1):
    cid = lax.axis_index("c")
    sid = lax.axis_index("s")
    wid = cid * NS + sid
    lo = sid * ROWS_PER_TILE

    @pl.when(cid == 0)
    def _():
        pltpu.sync_copy(table_hbm.at[pl.ds(lo, ROWS_PER_TILE)],
                        acc.at[pl.ds(lo, ROWS_PER_TILE)])

    @pl.when(cid != 0)
    def _():
        pltpu.sync_copy(zeros_hbm.at[pl.ds(lo, ROWS_PER_TILE)],
                        acc.at[pl.ds(lo, ROWS_PER_TILE)])

    plsc.subcore_barrier()

    @pl.loop(0, GROUPS)
    def _(g):
        pltpu.sync_copy(src_hbm.at[wid, g], srcv)
        pltpu.sync_copy(dst_hbm.at[wid, g], dstv)

        @pl.loop(0, GCH, step=2)
        def _(j):
            cp0 = pltpu.async_copy(table_hbm.at[srcv.at[j]], rows0, sem0)
            cp1 = pltpu.async_copy(table_hbm.at[srcv.at[j + 1]], rows1, sem1)
            cp0.wait()
            pltpu.sync_copy(rows0, acc.at[dstv.at[j]], add=True)
            cp1.wait()
            pltpu.sync_copy(rows1, acc.at[dstv.at[j + 1]], add=True)

    plsc.subcore_barrier()
    pltpu.sync_copy(acc.at[pl.ds(lo, ROWS_PER_TILE)],
                    out_hbm.at[cid, pl.ds(lo, ROWS_PER_TILE)])


def _edge_agg(table, zeros, src3, dst3):
    mesh = plsc.VectorSubcoreMesh(core_axis_name="c", subcore_axis_name="s")
    f = pl.kernel(
        _edge_agg_body,
        out_type=jax.ShapeDtypeStruct((NC, NP, D), jnp.float32),
        mesh=mesh,
        scratch_types=[
            pltpu.VMEM((GCH, CHUNK), jnp.int32),
            pltpu.VMEM((GCH, CHUNK), jnp.int32),
            pltpu.VMEM((CHUNK, D), jnp.float32),
            pltpu.VMEM((CHUNK, D), jnp.float32),
            pltpu.VMEM_SHARED((NP, D), jnp.float32),
            pltpu.SemaphoreType.DMA,
            pltpu.SemaphoreType.DMA,
        ],
    )
    return f(table, zeros, src3, dst3)


def _tc1_body(acc_ref, w_ref, b_ref, o_ref):
    h = acc_ref[0] + acc_ref[1]
    o_ref[...] = jnp.maximum(
        jnp.dot(h, w_ref[...], preferred_element_type=jnp.float32)
        + b_ref[...], 0.0)


def _tc2_body(acc_ref, w_ref, b_ref, gid_ref, lab_ref,
              emb_ref, proto_ref, sim_ref):
    h = acc_ref[0] + acc_ref[1]
    h2 = jnp.dot(h, w_ref[...], preferred_element_type=jnp.float32) + b_ref[...]
    gid = gid_ref[...]
    giota = lax.broadcasted_iota(jnp.int32, (G, NP), 0)
    onehot = (gid == giota).astype(jnp.float32)
    g_sum = jnp.dot(onehot, h2, preferred_element_type=jnp.float32)
    g_cnt = jnp.sum(onehot, axis=1, keepdims=True)
    emb = g_sum / jnp.maximum(g_cnt, 1.0)
    lab = lab_ref[...]
    ciota = lax.broadcasted_iota(jnp.int32, (C, G), 0)
    oh2 = (lab == ciota).astype(jnp.float32)
    p_sum = jnp.dot(oh2, emb, preferred_element_type=jnp.float32)
    p_cnt = jnp.sum(oh2, axis=1, keepdims=True)
    proto = p_sum / jnp.maximum(p_cnt, 1.0)
    qn = emb / (jnp.sqrt(jnp.sum(emb * emb, axis=1, keepdims=True)) + 1e-8)
    pn = proto / (jnp.sqrt(jnp.sum(proto * proto, axis=1, keepdims=True))
                  + 1e-8)
    emb_ref[...] = emb
    proto_ref[...] = proto
    sim_ref[...] = lax.dot_general(
        qn, pn, (((1,), (1,)), ((), ())),
        preferred_element_type=jnp.float32)


def kernel(x, edge_index, graph_ids, graph_labels, W1, b1, W2, b2):
    f32 = jnp.float32
    x_pad = jnp.zeros((NP, D), f32).at[:N].set(x)
    zeros = jnp.zeros((NP, D), f32)
    pad_ids = (N + jnp.arange(EP - E, dtype=jnp.int32) % (NP - N))
    src3 = jnp.concatenate([edge_index[0], pad_ids]).reshape(
        NW, GROUPS, GCH, CHUNK)
    dst3 = jnp.concatenate([edge_index[1], pad_ids]).reshape(
        NW, GROUPS, GCH, CHUNK)

    acc1 = _edge_agg(x_pad, zeros, src3, dst3)
    h1 = pl.pallas_call(
        _tc1_body,
        out_shape=jax.ShapeDtypeStruct((NP, D), f32),
    )(acc1, W1, b1.reshape(1, D))

    acc2 = _edge_agg(h1, zeros, src3, dst3)
    gid = jnp.full((1, NP), G, jnp.int32).at[0, :N].set(graph_ids)
    lab = graph_labels.reshape(1, G)
    embedded, prototypes, similarities = pl.pallas_call(
        _tc2_body,
        out_shape=(
            jax.ShapeDtypeStruct((G, D), f32),
            jax.ShapeDtypeStruct((C, D), f32),
            jax.ShapeDtypeStruct((G, C), f32),
        ),
    )(acc2, W2, b2.reshape(1, D), gid, lab)
    return (embedded, prototypes, similarities)


# speedup vs baseline: 10.2474x; 1.0810x over previous
import jax
import jax.numpy as jnp
from jax import lax
from jax.experimental import pallas as pl
from jax.experimental.pallas import tpu as pltpu
from jax.experimental.pallas import tpu_sc as plsc

N = 10000
E = 320000
D = 128
G = 200
C = 5

NC = 2
NS = 16
NW = NC * NS
CHUNK = 128
NP = 10240
EPT = 10240
GCH = 40
GROUPS = EPT // (GCH * CHUNK)
EP = NW * EPT
ROWS_PER_TILE = NP // NS


def _edge_agg_body(table_hbm, zeros_hbm, src_hbm, dst_hbm, out_hbm,
                   srcv, dstv, rows0, rows1, acc, gsem0, gsem1, ssem0, ssem1):
    cid = lax.axis_index("c")
    sid = lax.axis_index("s")
    wid = cid * NS + sid
    lo = sid * ROWS_PER_TILE

    @pl.when(cid == 0)
    def _():
        pltpu.sync_copy(table_hbm.at[pl.ds(lo, ROWS_PER_TILE)],
                        acc.at[pl.ds(lo, ROWS_PER_TILE)])

    @pl.when(cid != 0)
    def _():
        pltpu.sync_copy(zeros_hbm.at[pl.ds(lo, ROWS_PER_TILE)],
                        acc.at[pl.ds(lo, ROWS_PER_TILE)])

    plsc.subcore_barrier()

    @pl.loop(0, GROUPS)
    def _(g):
        pltpu.sync_copy(src_hbm.at[wid, g], srcv)
        pltpu.sync_copy(dst_hbm.at[wid, g], dstv)
        pltpu.async_copy(table_hbm.at[srcv.at[0]], rows0, gsem0)
        pltpu.async_copy(table_hbm.at[srcv.at[1]], rows1, gsem1)

        @pl.loop(0, GCH, step=2)
        def _(j):
            pltpu.make_async_copy(
                table_hbm.at[srcv.at[j]], rows0, gsem0).wait()
            pltpu.async_copy(rows0, acc.at[dstv.at[j]], ssem0, add=True)
            pltpu.make_async_copy(
                table_hbm.at[srcv.at[j + 1]], rows1, gsem1).wait()
            pltpu.async_copy(rows1, acc.at[dstv.at[j + 1]], ssem1, add=True)
            pltpu.make_async_copy(rows0, acc.at[dstv.at[j]], ssem0).wait()

            @pl.when(j + 2 < GCH)
            def _():
                pltpu.async_copy(table_hbm.at[srcv.at[j + 2]], rows0, gsem0)
            pltpu.make_async_copy(rows1, acc.at[dstv.at[j + 1]], ssem1).wait()

            @pl.when(j + 3 < GCH)
            def _():
                pltpu.async_copy(table_hbm.at[srcv.at[j + 3]], rows1, gsem1)

    plsc.subcore_barrier()
    pltpu.sync_copy(acc.at[pl.ds(lo, ROWS_PER_TILE)],
                    out_hbm.at[cid, pl.ds(lo, ROWS_PER_TILE)])


def _edge_agg(table, zeros, src3, dst3):
    mesh = plsc.VectorSubcoreMesh(core_axis_name="c", subcore_axis_name="s")
    f = pl.kernel(
        _edge_agg_body,
        out_type=jax.ShapeDtypeStruct((NC, NP, D), jnp.float32),
        mesh=mesh,
        scratch_types=[
            pltpu.VMEM((GCH, CHUNK), jnp.int32),
            pltpu.VMEM((GCH, CHUNK), jnp.int32),
            pltpu.VMEM((CHUNK, D), jnp.float32),
            pltpu.VMEM((CHUNK, D), jnp.float32),
            pltpu.VMEM_SHARED((NP, D), jnp.float32),
            pltpu.SemaphoreType.DMA,
            pltpu.SemaphoreType.DMA,
            pltpu.SemaphoreType.DMA,
            pltpu.SemaphoreType.DMA,
        ],
    )
    return f(table, zeros, src3, dst3)


def _tc1_body(acc_ref, w_ref, b_ref, o_ref):
    h = acc_ref[0] + acc_ref[1]
    o_ref[...] = jnp.maximum(
        jnp.dot(h, w_ref[...], preferred_element_type=jnp.float32)
        + b_ref[...], 0.0)


def _tc2_body(acc_ref, w_ref, b_ref, gid_ref, lab_ref,
              emb_ref, proto_ref, sim_ref):
    h = acc_ref[0] + acc_ref[1]
    h2 = jnp.dot(h, w_ref[...], preferred_element_type=jnp.float32) + b_ref[...]
    gid = gid_ref[...]
    giota = lax.broadcasted_iota(jnp.int32, (G, NP), 0)
    onehot = (gid == giota).astype(jnp.float32)
    g_sum = jnp.dot(onehot, h2, preferred_element_type=jnp.float32)
    g_cnt = jnp.sum(onehot, axis=1, keepdims=True)
    emb = g_sum / jnp.maximum(g_cnt, 1.0)
    lab = lab_ref[...]
    ciota = lax.broadcasted_iota(jnp.int32, (C, G), 0)
    oh2 = (lab == ciota).astype(jnp.float32)
    p_sum = jnp.dot(oh2, emb, preferred_element_type=jnp.float32)
    p_cnt = jnp.sum(oh2, axis=1, keepdims=True)
    proto = p_sum / jnp.maximum(p_cnt, 1.0)
    qn = emb / (jnp.sqrt(jnp.sum(emb * emb, axis=1, keepdims=True)) + 1e-8)
    pn = proto / (jnp.sqrt(jnp.sum(proto * proto, axis=1, keepdims=True))
                  + 1e-8)
    emb_ref[...] = emb
    proto_ref[...] = proto
    sim_ref[...] = lax.dot_general(
        qn, pn, (((1,), (1,)), ((), ())),
        preferred_element_type=jnp.float32)


def kernel(x, edge_index, graph_ids, graph_labels, W1, b1, W2, b2):
    f32 = jnp.float32
    x_pad = jnp.zeros((NP, D), f32).at[:N].set(x)
    zeros = jnp.zeros((NP, D), f32)
    pad_ids = (N + jnp.arange(EP - E, dtype=jnp.int32) % (NP - N))
    src3 = jnp.concatenate([edge_index[0], pad_ids]).reshape(
        NW, GROUPS, GCH, CHUNK)
    dst3 = jnp.concatenate([edge_index[1], pad_ids]).reshape(
        NW, GROUPS, GCH, CHUNK)

    acc1 = _edge_agg(x_pad, zeros, src3, dst3)
    h1 = pl.pallas_call(
        _tc1_body,
        out_shape=jax.ShapeDtypeStruct((NP, D), f32),
    )(acc1, W1, b1.reshape(1, D))

    acc2 = _edge_agg(h1, zeros, src3, dst3)
    gid = jnp.full((1, NP), G, jnp.int32).at[0, :N].set(graph_ids)
    lab = graph_labels.reshape(1, G)
    embedded, prototypes, similarities = pl.pallas_call(
        _tc2_body,
        out_shape=(
            jax.ShapeDtypeStruct((G, D), f32),
            jax.ShapeDtypeStruct((C, D), f32),
            jax.ShapeDtypeStruct((G, C), f32),
        ),
    )(acc2, W2, b2.reshape(1, D), gid, lab)
    return (embedded, prototypes, similarities)
